# Initial kernel scaffold; baseline (speedup 1.0000x reference)
import jax
import jax.numpy as jnp
from jax.experimental import pallas as pl


def kernel(x, edge_index, forecast_delta, t_net_w, t_net_b, adaln_w, adaln_b, qkv_w, qkv_b, proj_w, proj_b, mlp_w1, mlp_b1, mlp_w2, mlp_b2):
    raise NotImplementedError("write your pallas kernel here")


# speedup vs baseline: 31.8230x; 31.8230x over previous
import functools

import jax
import jax.numpy as jnp
import numpy as np
from jax import lax
from jax.experimental import pallas as pl
from jax.experimental.pallas import tpu as pltpu
from jax.experimental.pallas import tpu_sc as plsc

HIDDEN = 128
DEPTH = 4
HEADS = 8
DH = HIDDEN // HEADS
FREQ = 256
MLPD = 4 * HIDDEN

N_NODES = 10000
N_EDGES = 320000

BM = 2000
BE = 2000

NC = 2
NS = 16
NW = NC * NS
EPW = N_EDGES // NW
CH = 80
NIT = EPW // CH
M_PAD = 10240
ROWS_PER_TILE = M_PAD // NS


def _mod_body(fd_ref, tw_ref, tb_ref, aw_ref, ab_ref, out_ref):
    half = FREQ // 2
    t = fd_ref[0, 0]
    i = lax.broadcasted_iota(jnp.int32, (1, half), 1).astype(jnp.float32)
    freqs = jnp.exp(i * (-np.log(10000.0) / half))
    args = t * freqs
    emb = jnp.concatenate([jnp.cos(args), jnp.sin(args)], axis=1)
    temb = jnp.dot(emb, tw_ref[...], preferred_element_type=jnp.float32) + tb_ref[...]
    s = temb * jax.nn.sigmoid(temb)
    for l in range(DEPTH):
        row = jnp.dot(s, aw_ref[l], preferred_element_type=jnp.float32)
        out_ref[pl.ds(l, 1), :] = row + ab_ref[pl.ds(l, 1), :]


def _compute_mod(fd, t_net_w, t_net_b, adaln_w, adaln_b):
    return pl.pallas_call(
        _mod_body,
        out_shape=jax.ShapeDtypeStruct((DEPTH, 6 * HIDDEN), jnp.float32),
    )(fd.reshape(1, 1), t_net_w, t_net_b.reshape(1, HIDDEN), adaln_w, adaln_b)


def _qkv_body(h_ref, mod_ref, w_ref, b_ref, q_ref, k_ref, v_ref):
    h = h_ref[...]
    mu = jnp.mean(h, axis=1, keepdims=True)
    var = jnp.mean((h - mu) * (h - mu), axis=1, keepdims=True)
    hn = (h - mu) * lax.rsqrt(var + 1e-6)
    sh = mod_ref[0:1, 0:HIDDEN]
    sc = mod_ref[0:1, HIDDEN:2 * HIDDEN]
    hn = hn * (1.0 + sc) + sh
    qkv = jnp.dot(hn, w_ref[...], preferred_element_type=jnp.float32) + b_ref[...]
    q_ref[...] = qkv[:, 0:HIDDEN]
    k_ref[...] = qkv[:, HIDDEN:2 * HIDDEN]
    v_ref[...] = qkv[:, 2 * HIDDEN:3 * HIDDEN]


def _compute_qkv(h, mod_row, qkv_w, qkv_b):
    grid = (N_NODES // BM,)
    obs = pl.BlockSpec((BM, HIDDEN), lambda i: (i, 0))
    return pl.pallas_call(
        _qkv_body,
        grid=grid,
        in_specs=[
            pl.BlockSpec((BM, HIDDEN), lambda i: (i, 0)),
            pl.BlockSpec((1, 6 * HIDDEN), lambda i: (0, 0)),
            pl.BlockSpec((HIDDEN, 3 * HIDDEN), lambda i: (0, 0)),
            pl.BlockSpec((1, 3 * HIDDEN), lambda i: (0, 0)),
        ],
        out_specs=[obs, obs, obs],
        out_shape=[jax.ShapeDtypeStruct((N_NODES, HIDDEN), jnp.float32)] * 3,
    )(h, mod_row, qkv_w, qkv_b.reshape(1, 3 * HIDDEN))


@functools.lru_cache(maxsize=None)
def _sc_mesh():
    return plsc.VectorSubcoreMesh(
        core_axis_name="c", subcore_axis_name="s", num_cores=NC, num_subcores=NS)


def _gather_body(q_hbm, k_hbm, v_hbm, dst_hbm, src_hbm,
                 qd_hbm, ks_hbm, vs_hbm,
                 dsti, srci, qb, kb, vb, s1, s2, s3):
    wid = lax.axis_index("s") * NC + lax.axis_index("c")
    base0 = wid * EPW

    def body(i, carry):
        base = base0 + i * CH
        pltpu.sync_copy(dst_hbm.at[pl.ds(base, CH)], dsti)
        pltpu.sync_copy(src_hbm.at[pl.ds(base, CH)], srci)
        c1 = pltpu.async_copy(q_hbm.at[dsti], qb, s1)
        c2 = pltpu.async_copy(k_hbm.at[srci], kb, s2)
        c3 = pltpu.async_copy(v_hbm.at[srci], vb, s3)
        c1.wait()
        c2.wait()
        c3.wait()
        pltpu.sync_copy(qb, qd_hbm.at[pl.ds(base, CH)])
        pltpu.sync_copy(kb, ks_hbm.at[pl.ds(base, CH)])
        pltpu.sync_copy(vb, vs_hbm.at[pl.ds(base, CH)])
        return carry

    lax.fori_loop(0, NIT, body, 0)


@functools.lru_cache(maxsize=None)
def _build_gather():
  return pl.kernel(
    _gather_body,
    out_type=[jax.ShapeDtypeStruct((N_EDGES, HIDDEN), jnp.float32)] * 3,
    mesh=_sc_mesh(),
    scratch_types=[
        pltpu.VMEM((CH,), jnp.int32),
        pltpu.VMEM((CH,), jnp.int32),
        pltpu.VMEM((CH, HIDDEN), jnp.float32),
        pltpu.VMEM((CH, HIDDEN), jnp.float32),
        pltpu.VMEM((CH, HIDDEN), jnp.float32),
        pltpu.SemaphoreType.DMA,
        pltpu.SemaphoreType.DMA,
        pltpu.SemaphoreType.DMA,
    ],
  )


def _edge_body(qd_ref, ks_ref, vs_ref, S_ref, SB_ref, w_ref, e_ref):
    p = qd_ref[...] * ks_ref[...]
    score = jnp.dot(p, S_ref[...], preferred_element_type=jnp.float32) * (1.0 / np.sqrt(DH))
    e = jnp.exp(score)
    eb = jnp.dot(e, SB_ref[...], preferred_element_type=jnp.float32)
    w_ref[...] = vs_ref[...] * eb
    e_ref[...] = eb


def _compute_edge(qd, ks, vs, S, SB):
    grid = (N_EDGES // BE,)
    ebs = pl.BlockSpec((BE, HIDDEN), lambda i: (i, 0))
    return pl.pallas_call(
        _edge_body,
        grid=grid,
        in_specs=[
            ebs, ebs, ebs,
            pl.BlockSpec((HIDDEN, HEADS), lambda i: (0, 0)),
            pl.BlockSpec((HEADS, HIDDEN), lambda i: (0, 0)),
        ],
        out_specs=[ebs, ebs],
        out_shape=[jax.ShapeDtypeStruct((N_EDGES, HIDDEN), jnp.float32)] * 2,
    )(qd, ks, vs, S, SB)


def _scatter_body(w_hbm, dst_hbm, zw_hbm, pw_hbm, dsti, wb, acc_w):
    cid = lax.axis_index("c")
    sid = lax.axis_index("s")
    wid = sid * NC + cid
    row0 = sid * ROWS_PER_TILE
    pltpu.sync_copy(zw_hbm, acc_w.at[pl.ds(row0, ROWS_PER_TILE)])
    plsc.subcore_barrier()

    def body(i, carry):
        base = wid * EPW + i * CH
        pltpu.sync_copy(dst_hbm.at[pl.ds(base, CH)], dsti)
        pltpu.sync_copy(w_hbm.at[pl.ds(base, CH)], wb)
        pltpu.sync_copy(wb, acc_w.at[dsti], add=True)
        return carry

    lax.fori_loop(0, NIT, body, 0)
    plsc.subcore_barrier()
    pltpu.sync_copy(acc_w.at[pl.ds(row0, ROWS_PER_TILE)],
                    pw_hbm.at[cid, pl.ds(row0, ROWS_PER_TILE)])


@functools.lru_cache(maxsize=None)
def _build_scatter():
  return pl.kernel(
    _scatter_body,
    out_type=jax.ShapeDtypeStruct((NC, M_PAD, HIDDEN), jnp.float32),
    mesh=_sc_mesh(),
    scratch_types=[
        pltpu.VMEM((CH,), jnp.int32),
        pltpu.VMEM((CH, HIDDEN), jnp.float32),
        pltpu.VMEM_SHARED((M_PAD, HIDDEN), jnp.float32),
    ],
  )


def _out_body(h_ref, pw_ref, pe_ref, mod_ref,
              pjw_ref, pjb_ref, w1_ref, b1_ref, w2_ref, b2_ref, out_ref):
    num = pw_ref[0] + pw_ref[1]
    den = pe_ref[0] + pe_ref[1]
    msg = num / (den + 1e-9)
    attn = jnp.dot(msg, pjw_ref[...], preferred_element_type=jnp.float32) + pjb_ref[...]
    g_msa = mod_ref[0:1, 2 * HIDDEN:3 * HIDDEN]
    sh_mlp = mod_ref[0:1, 3 * HIDDEN:4 * HIDDEN]
    sc_mlp = mod_ref[0:1, 4 * HIDDEN:5 * HIDDEN]
    g_mlp = mod_ref[0:1, 5 * HIDDEN:6 * HIDDEN]
    h1 = h_ref[...] + g_msa * attn
    mu = jnp.mean(h1, axis=1, keepdims=True)
    var = jnp.mean((h1 - mu) * (h1 - mu), axis=1, keepdims=True)
    hm = (h1 - mu) * lax.rsqrt(var + 1e-6)
    hm = hm * (1.0 + sc_mlp) + sh_mlp
    z = jnp.dot(hm, w1_ref[...], preferred_element_type=jnp.float32) + b1_ref[...]
    t = 0.5 * z * (1.0 + lax.erf(z * np.float32(1.0 / np.sqrt(2.0))))
    mlp = jnp.dot(t, w2_ref[...], preferred_element_type=jnp.float32) + b2_ref[...]
    out_ref[...] = h1 + g_mlp * mlp


def _compute_out(h, pw, pe, mod_row, proj_w, proj_b, w1, b1, w2, b2):
    grid = (N_NODES // BM,)
    return pl.pallas_call(
        _out_body,
        grid=grid,
        in_specs=[
            pl.BlockSpec((BM, HIDDEN), lambda i: (i, 0)),
            pl.BlockSpec((NC, BM, HIDDEN), lambda i: (0, i, 0)),
            pl.BlockSpec((NC, BM, HIDDEN), lambda i: (0, i, 0)),
            pl.BlockSpec((1, 6 * HIDDEN), lambda i: (0, 0)),
            pl.BlockSpec((HIDDEN, HIDDEN), lambda i: (0, 0)),
            pl.BlockSpec((1, HIDDEN), lambda i: (0, 0)),
            pl.BlockSpec((HIDDEN, MLPD), lambda i: (0, 0)),
            pl.BlockSpec((1, MLPD), lambda i: (0, 0)),
            pl.BlockSpec((MLPD, HIDDEN), lambda i: (0, 0)),
            pl.BlockSpec((1, HIDDEN), lambda i: (0, 0)),
        ],
        out_specs=pl.BlockSpec((BM, HIDDEN), lambda i: (i, 0)),
        out_shape=jax.ShapeDtypeStruct((N_NODES, HIDDEN), jnp.float32),
    )(h, pw, pe, mod_row, proj_w, proj_b.reshape(1, HIDDEN),
      w1, b1.reshape(1, MLPD), w2, b2.reshape(1, HIDDEN))


def kernel(x, edge_index, forecast_delta, t_net_w, t_net_b, adaln_w, adaln_b,
           qkv_w, qkv_b, proj_w, proj_b, mlp_w1, mlp_b1, mlp_w2, mlp_b2):
    Bv, Nv, C = x.shape
    h = x.reshape(Bv * Nv, C)
    src = edge_index[0]
    dst = edge_index[1]

    lane = np.arange(HIDDEN) // DH
    S = jnp.asarray((lane[:, None] == np.arange(HEADS)[None, :]).astype(np.float32))
    SB = S.T
    zw = jnp.zeros((ROWS_PER_TILE, HIDDEN), jnp.float32)

    mod = _compute_mod(forecast_delta, t_net_w, t_net_b, adaln_w, adaln_b)

    for l in range(DEPTH):
        mod_row = mod[l:l + 1]
        q, k, v = _compute_qkv(h, mod_row, qkv_w[l], qkv_b[l])
        qd, ks, vs = _build_gather()(q, k, v, dst, src)
        w, eb = _compute_edge(qd, ks, vs, S, SB)
        pw = _build_scatter()(w, dst, zw)
        pe = _build_scatter()(eb, dst, zw)
        h = _compute_out(h, pw, pe, mod_row, proj_w[l], proj_b[l],
                         mlp_w1[l], mlp_b1[l], mlp_w2[l], mlp_b2[l])

    return h.reshape(Bv, Nv, C)


# speedup vs baseline: 37.4874x; 1.1780x over previous
import functools

import jax
import jax.numpy as jnp
import numpy as np
from jax import lax
from jax.experimental import pallas as pl
from jax.experimental.pallas import tpu as pltpu
from jax.experimental.pallas import tpu_sc as plsc

HIDDEN = 128
DEPTH = 4
HEADS = 8
DH = HIDDEN // HEADS
FREQ = 256
MLPD = 4 * HIDDEN

N_NODES = 10000
N_EDGES = 320000

BM = 2000
BE = 2000

NC = 2
NS = 16
NW = NC * NS
EPW = N_EDGES // NW
CH = 80
NIT = EPW // CH
M_PAD = 10240
ROWS_PER_TILE = M_PAD // NS


def _mod_body(fd_ref, tw_ref, tb_ref, aw_ref, ab_ref, out_ref):
    half = FREQ // 2
    t = fd_ref[0, 0]
    i = lax.broadcasted_iota(jnp.int32, (1, half), 1).astype(jnp.float32)
    freqs = jnp.exp(i * (-np.log(10000.0) / half))
    args = t * freqs
    emb = jnp.concatenate([jnp.cos(args), jnp.sin(args)], axis=1)
    temb = jnp.dot(emb, tw_ref[...], preferred_element_type=jnp.float32) + tb_ref[...]
    s = temb * jax.nn.sigmoid(temb)
    for l in range(DEPTH):
        row = jnp.dot(s, aw_ref[l], preferred_element_type=jnp.float32)
        out_ref[pl.ds(l, 1), :] = row + ab_ref[pl.ds(l, 1), :]


def _compute_mod(fd, t_net_w, t_net_b, adaln_w, adaln_b):
    return pl.pallas_call(
        _mod_body,
        out_shape=jax.ShapeDtypeStruct((DEPTH, 6 * HIDDEN), jnp.float32),
    )(fd.reshape(1, 1), t_net_w, t_net_b.reshape(1, HIDDEN), adaln_w, adaln_b)


def _qkv_body(h_ref, mod_ref, w_ref, b_ref, q_ref, k_ref, v_ref):
    h = h_ref[...]
    mu = jnp.mean(h, axis=1, keepdims=True)
    var = jnp.mean((h - mu) * (h - mu), axis=1, keepdims=True)
    hn = (h - mu) * lax.rsqrt(var + 1e-6)
    sh = mod_ref[0:1, 0:HIDDEN]
    sc = mod_ref[0:1, HIDDEN:2 * HIDDEN]
    hn = hn * (1.0 + sc) + sh
    qkv = jnp.dot(hn, w_ref[...], preferred_element_type=jnp.float32) + b_ref[...]
    q_ref[...] = qkv[:, 0:HIDDEN]
    k_ref[...] = qkv[:, HIDDEN:2 * HIDDEN]
    v_ref[...] = qkv[:, 2 * HIDDEN:3 * HIDDEN]


def _compute_qkv(h, mod_row, qkv_w, qkv_b):
    grid = (N_NODES // BM,)
    obs = pl.BlockSpec((BM, HIDDEN), lambda i: (i, 0))
    return pl.pallas_call(
        _qkv_body,
        grid=grid,
        in_specs=[
            pl.BlockSpec((BM, HIDDEN), lambda i: (i, 0)),
            pl.BlockSpec((1, 6 * HIDDEN), lambda i: (0, 0)),
            pl.BlockSpec((HIDDEN, 3 * HIDDEN), lambda i: (0, 0)),
            pl.BlockSpec((1, 3 * HIDDEN), lambda i: (0, 0)),
        ],
        out_specs=[obs, obs, obs],
        out_shape=[jax.ShapeDtypeStruct((N_NODES, HIDDEN), jnp.float32)] * 3,
    )(h, mod_row, qkv_w, qkv_b.reshape(1, 3 * HIDDEN))


@functools.lru_cache(maxsize=None)
def _sc_mesh():
    return plsc.VectorSubcoreMesh(
        core_axis_name="c", subcore_axis_name="s", num_cores=NC, num_subcores=NS)


def _gather_body(q_hbm, k_hbm, v_hbm, dst_hbm, src_hbm,
                 qd_hbm, ks_hbm, vs_hbm,
                 dsti, srci, qb, kb, vb, s1, s2, s3):
    wid = lax.axis_index("s") * NC + lax.axis_index("c")
    base0 = wid * EPW

    def body(i, carry):
        base = base0 + i * CH
        pltpu.sync_copy(dst_hbm.at[pl.ds(base, CH)], dsti)
        pltpu.sync_copy(src_hbm.at[pl.ds(base, CH)], srci)
        c1 = pltpu.async_copy(q_hbm.at[dsti], qb, s1)
        c2 = pltpu.async_copy(k_hbm.at[srci], kb, s2)
        c3 = pltpu.async_copy(v_hbm.at[srci], vb, s3)
        c1.wait()
        c2.wait()
        c3.wait()
        pltpu.sync_copy(qb, qd_hbm.at[pl.ds(base, CH)])
        pltpu.sync_copy(kb, ks_hbm.at[pl.ds(base, CH)])
        pltpu.sync_copy(vb, vs_hbm.at[pl.ds(base, CH)])
        return carry

    lax.fori_loop(0, NIT, body, 0)


@functools.lru_cache(maxsize=None)
def _build_gather():
  return pl.kernel(
    _gather_body,
    out_type=[jax.ShapeDtypeStruct((N_EDGES, HIDDEN), jnp.float32)] * 3,
    mesh=_sc_mesh(),
    scratch_types=[
        pltpu.VMEM((CH,), jnp.int32),
        pltpu.VMEM((CH,), jnp.int32),
        pltpu.VMEM((CH, HIDDEN), jnp.float32),
        pltpu.VMEM((CH, HIDDEN), jnp.float32),
        pltpu.VMEM((CH, HIDDEN), jnp.float32),
        pltpu.SemaphoreType.DMA,
        pltpu.SemaphoreType.DMA,
        pltpu.SemaphoreType.DMA,
    ],
  )


def _edge_body(qd_ref, ks_ref, vs_ref, S_ref, SB_ref, w_ref, e_ref):
    p = qd_ref[...] * ks_ref[...]
    score = jnp.dot(p, S_ref[...], preferred_element_type=jnp.float32) * (1.0 / np.sqrt(DH))
    e = jnp.exp(score)
    eb = jnp.dot(e, SB_ref[...], preferred_element_type=jnp.float32)
    w_ref[...] = vs_ref[...] * eb
    e_ref[...] = eb


def _compute_edge(qd, ks, vs, S, SB):
    grid = (N_EDGES // BE,)
    ebs = pl.BlockSpec((BE, HIDDEN), lambda i: (i, 0))
    return pl.pallas_call(
        _edge_body,
        grid=grid,
        in_specs=[
            ebs, ebs, ebs,
            pl.BlockSpec((HIDDEN, HEADS), lambda i: (0, 0)),
            pl.BlockSpec((HEADS, HIDDEN), lambda i: (0, 0)),
        ],
        out_specs=[ebs, ebs],
        out_shape=[jax.ShapeDtypeStruct((N_EDGES, HIDDEN), jnp.float32)] * 2,
    )(qd, ks, vs, S, SB)


def _scatter_body(w_hbm, dst_hbm, zw_hbm, pw_hbm,
                  dsti0, dsti1, wb0, wb1, si0, si1, sw0, sw1, acc_w):
    cid = lax.axis_index("c")
    sid = lax.axis_index("s")
    wid = sid * NC + cid
    row0 = sid * ROWS_PER_TILE
    base0 = wid * EPW
    dsti = (dsti0, dsti1)
    wb = (wb0, wb1)
    si = (si0, si1)
    sw = (sw0, sw1)

    def start(i, b):
        pltpu.async_copy(dst_hbm.at[pl.ds(base0 + i * CH, CH)], dsti[b], si[b])
        pltpu.async_copy(w_hbm.at[pl.ds(base0 + i * CH, CH)], wb[b], sw[b])

    def wait(i, b):
        pltpu.make_async_copy(dst_hbm.at[pl.ds(base0 + i * CH, CH)],
                              dsti[b], si[b]).wait()
        pltpu.make_async_copy(w_hbm.at[pl.ds(base0 + i * CH, CH)],
                              wb[b], sw[b]).wait()

    pltpu.sync_copy(zw_hbm, acc_w.at[pl.ds(row0, ROWS_PER_TILE)])
    plsc.subcore_barrier()

    start(0, 0)

    def body(g, carry):
        i0 = 2 * g
        wait(i0, 0)
        start(i0 + 1, 1)
        pltpu.sync_copy(wb[0], acc_w.at[dsti[0]], add=True)
        wait(i0 + 1, 1)

        @pl.when(i0 + 2 < NIT)
        def _():
            start(i0 + 2, 0)

        pltpu.sync_copy(wb[1], acc_w.at[dsti[1]], add=True)
        return carry

    lax.fori_loop(0, NIT // 2, body, 0)
    wait(NIT - 1, 0)
    pltpu.sync_copy(wb[0], acc_w.at[dsti[0]], add=True)

    plsc.subcore_barrier()
    pltpu.sync_copy(acc_w.at[pl.ds(row0, ROWS_PER_TILE)],
                    pw_hbm.at[cid, pl.ds(row0, ROWS_PER_TILE)])


@functools.lru_cache(maxsize=None)
def _build_scatter():
  return pl.kernel(
    _scatter_body,
    out_type=jax.ShapeDtypeStruct((NC, M_PAD, HIDDEN), jnp.float32),
    mesh=_sc_mesh(),
    scratch_types=[
        pltpu.VMEM((CH,), jnp.int32),
        pltpu.VMEM((CH,), jnp.int32),
        pltpu.VMEM((CH, HIDDEN), jnp.float32),
        pltpu.VMEM((CH, HIDDEN), jnp.float32),
        pltpu.SemaphoreType.DMA,
        pltpu.SemaphoreType.DMA,
        pltpu.SemaphoreType.DMA,
        pltpu.SemaphoreType.DMA,
        pltpu.VMEM_SHARED((M_PAD, HIDDEN), jnp.float32),
    ],
  )


def _out_body(h_ref, pw_ref, pe_ref, mod_ref,
              pjw_ref, pjb_ref, w1_ref, b1_ref, w2_ref, b2_ref, out_ref):
    num = pw_ref[0] + pw_ref[1]
    den = pe_ref[0] + pe_ref[1]
    msg = num / (den + 1e-9)
    attn = jnp.dot(msg, pjw_ref[...], preferred_element_type=jnp.float32) + pjb_ref[...]
    g_msa = mod_ref[0:1, 2 * HIDDEN:3 * HIDDEN]
    sh_mlp = mod_ref[0:1, 3 * HIDDEN:4 * HIDDEN]
    sc_mlp = mod_ref[0:1, 4 * HIDDEN:5 * HIDDEN]
    g_mlp = mod_ref[0:1, 5 * HIDDEN:6 * HIDDEN]
    h1 = h_ref[...] + g_msa * attn
    mu = jnp.mean(h1, axis=1, keepdims=True)
    var = jnp.mean((h1 - mu) * (h1 - mu), axis=1, keepdims=True)
    hm = (h1 - mu) * lax.rsqrt(var + 1e-6)
    hm = hm * (1.0 + sc_mlp) + sh_mlp
    z = jnp.dot(hm, w1_ref[...], preferred_element_type=jnp.float32) + b1_ref[...]
    t = 0.5 * z * (1.0 + lax.erf(z * np.float32(1.0 / np.sqrt(2.0))))
    mlp = jnp.dot(t, w2_ref[...], preferred_element_type=jnp.float32) + b2_ref[...]
    out_ref[...] = h1 + g_mlp * mlp


def _compute_out(h, pw, pe, mod_row, proj_w, proj_b, w1, b1, w2, b2):
    grid = (N_NODES // BM,)
    return pl.pallas_call(
        _out_body,
        grid=grid,
        in_specs=[
            pl.BlockSpec((BM, HIDDEN), lambda i: (i, 0)),
            pl.BlockSpec((NC, BM, HIDDEN), lambda i: (0, i, 0)),
            pl.BlockSpec((NC, BM, HIDDEN), lambda i: (0, i, 0)),
            pl.BlockSpec((1, 6 * HIDDEN), lambda i: (0, 0)),
            pl.BlockSpec((HIDDEN, HIDDEN), lambda i: (0, 0)),
            pl.BlockSpec((1, HIDDEN), lambda i: (0, 0)),
            pl.BlockSpec((HIDDEN, MLPD), lambda i: (0, 0)),
            pl.BlockSpec((1, MLPD), lambda i: (0, 0)),
            pl.BlockSpec((MLPD, HIDDEN), lambda i: (0, 0)),
            pl.BlockSpec((1, HIDDEN), lambda i: (0, 0)),
        ],
        out_specs=pl.BlockSpec((BM, HIDDEN), lambda i: (i, 0)),
        out_shape=jax.ShapeDtypeStruct((N_NODES, HIDDEN), jnp.float32),
    )(h, pw, pe, mod_row, proj_w, proj_b.reshape(1, HIDDEN),
      w1, b1.reshape(1, MLPD), w2, b2.reshape(1, HIDDEN))


def kernel(x, edge_index, forecast_delta, t_net_w, t_net_b, adaln_w, adaln_b,
           qkv_w, qkv_b, proj_w, proj_b, mlp_w1, mlp_b1, mlp_w2, mlp_b2):
    Bv, Nv, C = x.shape
    h = x.reshape(Bv * Nv, C)
    src = edge_index[0]
    dst = edge_index[1]

    lane = np.arange(HIDDEN) // DH
    S = jnp.asarray((lane[:, None] == np.arange(HEADS)[None, :]).astype(np.float32))
    SB = S.T
    zw = jnp.zeros((ROWS_PER_TILE, HIDDEN), jnp.float32)

    mod = _compute_mod(forecast_delta, t_net_w, t_net_b, adaln_w, adaln_b)

    for l in range(DEPTH):
        mod_row = mod[l:l + 1]
        q, k, v = _compute_qkv(h, mod_row, qkv_w[l], qkv_b[l])
        qd, ks, vs = _build_gather()(q, k, v, dst, src)
        w, eb = _compute_edge(qd, ks, vs, S, SB)
        pw = _build_scatter()(w, dst, zw)
        pe = _build_scatter()(eb, dst, zw)
        h = _compute_out(h, pw, pe, mod_row, proj_w[l], proj_b[l],
                         mlp_w1[l], mlp_b1[l], mlp_w2[l], mlp_b2[l])

    return h.reshape(Bv, Nv, C)


# speedup vs baseline: 37.5437x; 1.0015x over previous
import functools

import jax
import jax.numpy as jnp
import numpy as np
from jax import lax
from jax.experimental import pallas as pl
from jax.experimental.pallas import tpu as pltpu
from jax.experimental.pallas import tpu_sc as plsc

HIDDEN = 128
DEPTH = 4
HEADS = 8
DH = HIDDEN // HEADS
FREQ = 256
MLPD = 4 * HIDDEN

N_NODES = 10000
N_EDGES = 320000

BM = 2000
BE = 2000

NC = 2
NS = 16
NW = NC * NS
EPW = N_EDGES // NW
CH = 80
NIT = EPW // CH
M_PAD = 10240
ROWS_PER_TILE = M_PAD // NS


def _mod_body(fd_ref, tw_ref, tb_ref, aw_ref, ab_ref, out_ref):
    half = FREQ // 2
    t = fd_ref[0, 0]
    i = lax.broadcasted_iota(jnp.int32, (1, half), 1).astype(jnp.float32)
    freqs = jnp.exp(i * (-np.log(10000.0) / half))
    args = t * freqs
    emb = jnp.concatenate([jnp.cos(args), jnp.sin(args)], axis=1)
    temb = jnp.dot(emb, tw_ref[...], preferred_element_type=jnp.float32) + tb_ref[...]
    s = temb * jax.nn.sigmoid(temb)
    for l in range(DEPTH):
        row = jnp.dot(s, aw_ref[l], preferred_element_type=jnp.float32)
        out_ref[pl.ds(l, 1), :] = row + ab_ref[pl.ds(l, 1), :]


def _compute_mod(fd, t_net_w, t_net_b, adaln_w, adaln_b):
    return pl.pallas_call(
        _mod_body,
        out_shape=jax.ShapeDtypeStruct((DEPTH, 6 * HIDDEN), jnp.float32),
    )(fd.reshape(1, 1), t_net_w, t_net_b.reshape(1, HIDDEN), adaln_w, adaln_b)


def _qkv_body(h_ref, mod_ref, w_ref, b_ref, q_ref, k_ref, v_ref):
    h = h_ref[...]
    mu = jnp.mean(h, axis=1, keepdims=True)
    var = jnp.mean((h - mu) * (h - mu), axis=1, keepdims=True)
    hn = (h - mu) * lax.rsqrt(var + 1e-6)
    sh = mod_ref[0:1, 0:HIDDEN]
    sc = mod_ref[0:1, HIDDEN:2 * HIDDEN]
    hn = hn * (1.0 + sc) + sh
    qkv = jnp.dot(hn, w_ref[...], preferred_element_type=jnp.float32) + b_ref[...]
    q_ref[...] = qkv[:, 0:HIDDEN]
    k_ref[...] = qkv[:, HIDDEN:2 * HIDDEN]
    v_ref[...] = qkv[:, 2 * HIDDEN:3 * HIDDEN]


def _compute_qkv(h, mod_row, qkv_w, qkv_b):
    grid = (N_NODES // BM,)
    obs = pl.BlockSpec((BM, HIDDEN), lambda i: (i, 0))
    return pl.pallas_call(
        _qkv_body,
        grid=grid,
        in_specs=[
            pl.BlockSpec((BM, HIDDEN), lambda i: (i, 0)),
            pl.BlockSpec((1, 6 * HIDDEN), lambda i: (0, 0)),
            pl.BlockSpec((HIDDEN, 3 * HIDDEN), lambda i: (0, 0)),
            pl.BlockSpec((1, 3 * HIDDEN), lambda i: (0, 0)),
        ],
        out_specs=[obs, obs, obs],
        out_shape=[jax.ShapeDtypeStruct((N_NODES, HIDDEN), jnp.float32)] * 3,
    )(h, mod_row, qkv_w, qkv_b.reshape(1, 3 * HIDDEN))


@functools.lru_cache(maxsize=None)
def _sc_mesh():
    return plsc.VectorSubcoreMesh(
        core_axis_name="c", subcore_axis_name="s", num_cores=NC, num_subcores=NS)


def _gather_body(q_hbm, k_hbm, v_hbm, dst_hbm, src_hbm,
                 qd_hbm, ks_hbm, vs_hbm,
                 dsti, srci, qb, kb, vb, s1, s2, s3):
    wid = lax.axis_index("s") * NC + lax.axis_index("c")
    base0 = wid * EPW

    def body(i, carry):
        base = base0 + i * CH
        pltpu.sync_copy(dst_hbm.at[pl.ds(base, CH)], dsti)
        pltpu.sync_copy(src_hbm.at[pl.ds(base, CH)], srci)
        c1 = pltpu.async_copy(q_hbm.at[dsti], qb, s1)
        c2 = pltpu.async_copy(k_hbm.at[srci], kb, s2)
        c3 = pltpu.async_copy(v_hbm.at[srci], vb, s3)
        c1.wait()
        c2.wait()
        c3.wait()
        pltpu.sync_copy(qb, qd_hbm.at[pl.ds(base, CH)])
        pltpu.sync_copy(kb, ks_hbm.at[pl.ds(base, CH)])
        pltpu.sync_copy(vb, vs_hbm.at[pl.ds(base, CH)])
        return carry

    lax.fori_loop(0, NIT, body, 0)


@functools.lru_cache(maxsize=None)
def _build_gather():
  return pl.kernel(
    _gather_body,
    out_type=[jax.ShapeDtypeStruct((N_EDGES, HIDDEN), jnp.float32)] * 3,
    mesh=_sc_mesh(),
    scratch_types=[
        pltpu.VMEM((CH,), jnp.int32),
        pltpu.VMEM((CH,), jnp.int32),
        pltpu.VMEM((CH, HIDDEN), jnp.float32),
        pltpu.VMEM((CH, HIDDEN), jnp.float32),
        pltpu.VMEM((CH, HIDDEN), jnp.float32),
        pltpu.SemaphoreType.DMA,
        pltpu.SemaphoreType.DMA,
        pltpu.SemaphoreType.DMA,
    ],
  )


def _edge_body(qd_ref, ks_ref, vs_ref, S_ref, SB_ref, w_ref, e_ref):
    p = qd_ref[...] * ks_ref[...]
    score = jnp.dot(p, S_ref[...], preferred_element_type=jnp.float32) * (1.0 / np.sqrt(DH))
    e = jnp.exp(score)
    eb = jnp.dot(e, SB_ref[...], preferred_element_type=jnp.float32)
    w_ref[...] = vs_ref[...] * eb
    e_ref[...] = eb


def _compute_edge(qd, ks, vs, S, SB):
    grid = (N_EDGES // BE,)
    ebs = pl.BlockSpec((BE, HIDDEN), lambda i: (i, 0))
    return pl.pallas_call(
        _edge_body,
        grid=grid,
        in_specs=[
            ebs, ebs, ebs,
            pl.BlockSpec((HIDDEN, HEADS), lambda i: (0, 0)),
            pl.BlockSpec((HEADS, HIDDEN), lambda i: (0, 0)),
        ],
        out_specs=[ebs, ebs],
        out_shape=[jax.ShapeDtypeStruct((N_EDGES, HIDDEN), jnp.float32)] * 2,
    )(qd, ks, vs, S, SB)


def _scatter_body(w_hbm, dst_hbm, zw_hbm, pw_hbm,
                  dsti0, dsti1, wb0, wb1,
                  si0, si1, sw0, sw1, acc_w):
    cid = lax.axis_index("c")
    sid = lax.axis_index("s")
    wid = sid * NC + cid
    row0 = sid * ROWS_PER_TILE
    base0 = wid * EPW
    dsti = (dsti0, dsti1)
    wb = (wb0, wb1)
    si = (si0, si1)
    sw = (sw0, sw1)

    def start(i, b):
        pltpu.async_copy(dst_hbm.at[pl.ds(base0 + i * CH, CH)], dsti[b], si[b])
        pltpu.async_copy(w_hbm.at[pl.ds(base0 + i * CH, CH)], wb[b], sw[b])

    def wait(i, b):
        pltpu.make_async_copy(dst_hbm.at[pl.ds(base0 + i * CH, CH)],
                              dsti[b], si[b]).wait()
        pltpu.make_async_copy(w_hbm.at[pl.ds(base0 + i * CH, CH)],
                              wb[b], sw[b]).wait()

    def scatter(b):
        pltpu.sync_copy(wb[b], acc_w.at[dsti[b]], add=True)

    pltpu.sync_copy(zw_hbm, acc_w.at[pl.ds(row0, ROWS_PER_TILE)])
    plsc.subcore_barrier()

    start(0, 0)

    def body(g, carry):
        i0 = 2 * g
        wait(i0, 0)
        start(i0 + 1, 1)
        scatter(0)
        wait(i0 + 1, 1)

        @pl.when(i0 + 2 < NIT)
        def _():
            start(i0 + 2, 0)

        scatter(1)
        return carry

    lax.fori_loop(0, NIT // 2, body, 0)
    wait(NIT - 1, 0)
    scatter(0)

    plsc.subcore_barrier()
    pltpu.sync_copy(acc_w.at[pl.ds(row0, ROWS_PER_TILE)],
                    pw_hbm.at[cid, pl.ds(row0, ROWS_PER_TILE)])


@functools.lru_cache(maxsize=None)
def _build_scatter():
  return pl.kernel(
    _scatter_body,
    out_type=jax.ShapeDtypeStruct((NC, M_PAD, HIDDEN), jnp.float32),
    mesh=_sc_mesh(),
    scratch_types=[
        pltpu.VMEM((CH,), jnp.int32),
        pltpu.VMEM((CH,), jnp.int32),
        pltpu.VMEM((CH, HIDDEN), jnp.float32),
        pltpu.VMEM((CH, HIDDEN), jnp.float32),
        pltpu.SemaphoreType.DMA,
        pltpu.SemaphoreType.DMA,
        pltpu.SemaphoreType.DMA,
        pltpu.SemaphoreType.DMA,
        pltpu.VMEM_SHARED((M_PAD, HIDDEN), jnp.float32),
    ],
  )


def _out_body(h_ref, pw_ref, pe_ref, mod_ref,
              pjw_ref, pjb_ref, w1_ref, b1_ref, w2_ref, b2_ref, out_ref):
    num = pw_ref[0].astype(jnp.float32) + pw_ref[1].astype(jnp.float32)
    den = pe_ref[0].astype(jnp.float32) + pe_ref[1].astype(jnp.float32)
    msg = num / (den + 1e-9)
    attn = jnp.dot(msg, pjw_ref[...], preferred_element_type=jnp.float32) + pjb_ref[...]
    g_msa = mod_ref[0:1, 2 * HIDDEN:3 * HIDDEN]
    sh_mlp = mod_ref[0:1, 3 * HIDDEN:4 * HIDDEN]
    sc_mlp = mod_ref[0:1, 4 * HIDDEN:5 * HIDDEN]
    g_mlp = mod_ref[0:1, 5 * HIDDEN:6 * HIDDEN]
    h1 = h_ref[...] + g_msa * attn
    mu = jnp.mean(h1, axis=1, keepdims=True)
    var = jnp.mean((h1 - mu) * (h1 - mu), axis=1, keepdims=True)
    hm = (h1 - mu) * lax.rsqrt(var + 1e-6)
    hm = hm * (1.0 + sc_mlp) + sh_mlp
    z = jnp.dot(hm, w1_ref[...], preferred_element_type=jnp.float32) + b1_ref[...]
    t = 0.5 * z * (1.0 + lax.erf(z * np.float32(1.0 / np.sqrt(2.0))))
    mlp = jnp.dot(t, w2_ref[...], preferred_element_type=jnp.float32) + b2_ref[...]
    out_ref[...] = h1 + g_mlp * mlp


def _compute_out(h, pw, pe, mod_row, proj_w, proj_b, w1, b1, w2, b2):
    grid = (N_NODES // BM,)
    return pl.pallas_call(
        _out_body,
        grid=grid,
        in_specs=[
            pl.BlockSpec((BM, HIDDEN), lambda i: (i, 0)),
            pl.BlockSpec((NC, BM, HIDDEN), lambda i: (0, i, 0)),
            pl.BlockSpec((NC, BM, HIDDEN), lambda i: (0, i, 0)),
            pl.BlockSpec((1, 6 * HIDDEN), lambda i: (0, 0)),
            pl.BlockSpec((HIDDEN, HIDDEN), lambda i: (0, 0)),
            pl.BlockSpec((1, HIDDEN), lambda i: (0, 0)),
            pl.BlockSpec((HIDDEN, MLPD), lambda i: (0, 0)),
            pl.BlockSpec((1, MLPD), lambda i: (0, 0)),
            pl.BlockSpec((MLPD, HIDDEN), lambda i: (0, 0)),
            pl.BlockSpec((1, HIDDEN), lambda i: (0, 0)),
        ],
        out_specs=pl.BlockSpec((BM, HIDDEN), lambda i: (i, 0)),
        out_shape=jax.ShapeDtypeStruct((N_NODES, HIDDEN), jnp.float32),
    )(h, pw, pe, mod_row, proj_w, proj_b.reshape(1, HIDDEN),
      w1, b1.reshape(1, MLPD), w2, b2.reshape(1, HIDDEN))


def kernel(x, edge_index, forecast_delta, t_net_w, t_net_b, adaln_w, adaln_b,
           qkv_w, qkv_b, proj_w, proj_b, mlp_w1, mlp_b1, mlp_w2, mlp_b2):
    Bv, Nv, C = x.shape
    h = x.reshape(Bv * Nv, C)
    src = edge_index[0]
    dst = edge_index[1]

    lane = np.arange(HIDDEN) // DH
    S = jnp.asarray((lane[:, None] == np.arange(HEADS)[None, :]).astype(np.float32))
    SB = S.T
    zw = jnp.zeros((ROWS_PER_TILE, HIDDEN), jnp.float32)

    mod = _compute_mod(forecast_delta, t_net_w, t_net_b, adaln_w, adaln_b)

    for l in range(DEPTH):
        mod_row = mod[l:l + 1]
        q, k, v = _compute_qkv(h, mod_row, qkv_w[l], qkv_b[l])
        qd, ks, vs = _build_gather()(q, k, v, dst, src)
        w, eb = _compute_edge(qd, ks, vs, S, SB)
        pw = _build_scatter()(w, dst, zw)
        pe = _build_scatter()(eb, dst, zw)
        h = _compute_out(h, pw, pe, mod_row, proj_w[l], proj_b[l],
                         mlp_w1[l], mlp_b1[l], mlp_w2[l], mlp_b2[l])

    return h.reshape(Bv, Nv, C)


# speedup vs baseline: 45.0046x; 1.1987x over previous
import functools

import jax
import jax.numpy as jnp
import numpy as np
from jax import lax
from jax.experimental import pallas as pl
from jax.experimental.pallas import tpu as pltpu
from jax.experimental.pallas import tpu_sc as plsc

HIDDEN = 128
DEPTH = 4
HEADS = 8
DH = HIDDEN // HEADS
FREQ = 256
MLPD = 4 * HIDDEN

N_NODES = 10000
N_EDGES = 320000

BM = 2000
BE = 2000

NC = 2
NS = 16
NW = NC * NS
EPW = N_EDGES // NW
CH = 80
NIT = EPW // CH
M_PAD = 10240
ROWS_PER_TILE = M_PAD // NS


def _mod_body(fd_ref, tw_ref, tb_ref, aw_ref, ab_ref, out_ref):
    half = FREQ // 2
    t = fd_ref[0, 0]
    i = lax.broadcasted_iota(jnp.int32, (1, half), 1).astype(jnp.float32)
    freqs = jnp.exp(i * (-np.log(10000.0) / half))
    args = t * freqs
    emb = jnp.concatenate([jnp.cos(args), jnp.sin(args)], axis=1)
    temb = jnp.dot(emb, tw_ref[...], preferred_element_type=jnp.float32) + tb_ref[...]
    s = temb * jax.nn.sigmoid(temb)
    for l in range(DEPTH):
        row = jnp.dot(s, aw_ref[l], preferred_element_type=jnp.float32)
        out_ref[pl.ds(l, 1), :] = row + ab_ref[pl.ds(l, 1), :]


def _compute_mod(fd, t_net_w, t_net_b, adaln_w, adaln_b):
    return pl.pallas_call(
        _mod_body,
        out_shape=jax.ShapeDtypeStruct((DEPTH, 6 * HIDDEN), jnp.float32),
    )(fd.reshape(1, 1), t_net_w, t_net_b.reshape(1, HIDDEN), adaln_w, adaln_b)


def _qkv_body(h_ref, mod_ref, w_ref, b_ref, q_ref, k_ref, v_ref):
    h = h_ref[...]
    mu = jnp.mean(h, axis=1, keepdims=True)
    var = jnp.mean((h - mu) * (h - mu), axis=1, keepdims=True)
    hn = (h - mu) * lax.rsqrt(var + 1e-6)
    sh = mod_ref[0:1, 0:HIDDEN]
    sc = mod_ref[0:1, HIDDEN:2 * HIDDEN]
    hn = hn * (1.0 + sc) + sh
    qkv = jnp.dot(hn, w_ref[...], preferred_element_type=jnp.float32) + b_ref[...]
    q_ref[...] = qkv[:, 0:HIDDEN]
    k_ref[...] = qkv[:, HIDDEN:2 * HIDDEN]
    v_ref[...] = qkv[:, 2 * HIDDEN:3 * HIDDEN]


def _compute_qkv(h, mod_row, qkv_w, qkv_b):
    grid = (N_NODES // BM,)
    obs = pl.BlockSpec((BM, HIDDEN), lambda i: (i, 0))
    return pl.pallas_call(
        _qkv_body,
        grid=grid,
        in_specs=[
            pl.BlockSpec((BM, HIDDEN), lambda i: (i, 0)),
            pl.BlockSpec((1, 6 * HIDDEN), lambda i: (0, 0)),
            pl.BlockSpec((HIDDEN, 3 * HIDDEN), lambda i: (0, 0)),
            pl.BlockSpec((1, 3 * HIDDEN), lambda i: (0, 0)),
        ],
        out_specs=[obs, obs, obs],
        out_shape=[jax.ShapeDtypeStruct((N_NODES, HIDDEN), jnp.float32)] * 3,
    )(h, mod_row, qkv_w, qkv_b.reshape(1, 3 * HIDDEN))


@functools.lru_cache(maxsize=None)
def _sc_mesh():
    return plsc.VectorSubcoreMesh(
        core_axis_name="c", subcore_axis_name="s", num_cores=NC, num_subcores=NS)


def _gather_body(q_hbm, k_hbm, v_hbm, dst_hbm, src_hbm,
                 qd_hbm, ks_hbm, vs_hbm,
                 di0, di1, sri0, sri1, qb0, qb1, kb0, kb1, vb0, vb1,
                 sdi0, sdi1, ssi0, ssi1,
                 sgq0, sgq1, sgk0, sgk1, sgv0, sgv1,
                 soq0, soq1, sok0, sok1, sov0, sov1):
    wid = lax.axis_index("s") * NC + lax.axis_index("c")
    base0 = wid * EPW
    di = (di0, di1)
    sri = (sri0, sri1)
    qb = (qb0, qb1)
    kb = (kb0, kb1)
    vb = (vb0, vb1)
    sdi = (sdi0, sdi1)
    ssi = (ssi0, ssi1)
    sgq = (sgq0, sgq1)
    sgk = (sgk0, sgk1)
    sgv = (sgv0, sgv1)
    soq = (soq0, soq1)
    sok = (sok0, sok1)
    sov = (sov0, sov1)

    def idx_start(i, b):
        pltpu.async_copy(dst_hbm.at[pl.ds(base0 + i * CH, CH)], di[b], sdi[b])
        pltpu.async_copy(src_hbm.at[pl.ds(base0 + i * CH, CH)], sri[b], ssi[b])

    def idx_wait(i, b):
        pltpu.make_async_copy(dst_hbm.at[pl.ds(base0 + i * CH, CH)],
                              di[b], sdi[b]).wait()
        pltpu.make_async_copy(src_hbm.at[pl.ds(base0 + i * CH, CH)],
                              sri[b], ssi[b]).wait()

    def g_start(b):
        pltpu.async_copy(q_hbm.at[di[b]], qb[b], sgq[b])
        pltpu.async_copy(k_hbm.at[sri[b]], kb[b], sgk[b])
        pltpu.async_copy(v_hbm.at[sri[b]], vb[b], sgv[b])

    def g_wait(b):
        pltpu.make_async_copy(q_hbm.at[di[b]], qb[b], sgq[b]).wait()
        pltpu.make_async_copy(k_hbm.at[sri[b]], kb[b], sgk[b]).wait()
        pltpu.make_async_copy(v_hbm.at[sri[b]], vb[b], sgv[b]).wait()

    def o_start(i, b):
        pltpu.async_copy(qb[b], qd_hbm.at[pl.ds(base0 + i * CH, CH)], soq[b])
        pltpu.async_copy(kb[b], ks_hbm.at[pl.ds(base0 + i * CH, CH)], sok[b])
        pltpu.async_copy(vb[b], vs_hbm.at[pl.ds(base0 + i * CH, CH)], sov[b])

    def o_wait(i, b):
        pltpu.make_async_copy(qb[b], qd_hbm.at[pl.ds(base0 + i * CH, CH)],
                              soq[b]).wait()
        pltpu.make_async_copy(kb[b], ks_hbm.at[pl.ds(base0 + i * CH, CH)],
                              sok[b]).wait()
        pltpu.make_async_copy(vb[b], vs_hbm.at[pl.ds(base0 + i * CH, CH)],
                              sov[b]).wait()

    idx_start(0, 0)
    idx_start(1, 1)
    idx_wait(0, 0)
    g_start(0)

    def body(g, carry):
        i0 = 2 * g
        g_wait(0)
        o_start(i0, 0)
        idx_wait(i0 + 1, 1)

        @pl.when(i0 > 0)
        def _():
            o_wait(i0 - 1, 1)

        g_start(1)

        @pl.when(i0 + 2 < NIT)
        def _():
            idx_start(i0 + 2, 0)

        g_wait(1)

        @pl.when(i0 + 3 < NIT)
        def _():
            idx_start(i0 + 3, 1)

        @pl.when(i0 + 2 < NIT)
        def _():
            idx_wait(i0 + 2, 0)
            o_wait(i0, 0)
            g_start(0)

        o_start(i0 + 1, 1)
        return carry

    lax.fori_loop(0, NIT // 2, body, 0)
    g_wait(0)
    o_wait(NIT - 2, 1)
    o_start(NIT - 1, 0)
    o_wait(NIT - 1, 0)


@functools.lru_cache(maxsize=None)
def _build_gather():
  return pl.kernel(
    _gather_body,
    out_type=[jax.ShapeDtypeStruct((N_EDGES, HIDDEN), jnp.float32)] * 3,
    mesh=_sc_mesh(),
    scratch_types=[
        pltpu.VMEM((CH,), jnp.int32),
        pltpu.VMEM((CH,), jnp.int32),
        pltpu.VMEM((CH,), jnp.int32),
        pltpu.VMEM((CH,), jnp.int32),
        pltpu.VMEM((CH, HIDDEN), jnp.float32),
        pltpu.VMEM((CH, HIDDEN), jnp.float32),
        pltpu.VMEM((CH, HIDDEN), jnp.float32),
        pltpu.VMEM((CH, HIDDEN), jnp.float32),
        pltpu.VMEM((CH, HIDDEN), jnp.float32),
        pltpu.VMEM((CH, HIDDEN), jnp.float32),
    ] + [pltpu.SemaphoreType.DMA] * 16,
  )


def _edge_body(qd_ref, ks_ref, vs_ref, S_ref, SB_ref, w_ref, e_ref):
    p = qd_ref[...] * ks_ref[...]
    score = jnp.dot(p, S_ref[...], preferred_element_type=jnp.float32) * (1.0 / np.sqrt(DH))
    e = jnp.exp(score)
    eb = jnp.dot(e, SB_ref[...], preferred_element_type=jnp.float32)
    w_ref[...] = vs_ref[...] * eb
    e_ref[...] = eb


def _compute_edge(qd, ks, vs, S, SB):
    grid = (N_EDGES // BE,)
    ebs = pl.BlockSpec((BE, HIDDEN), lambda i: (i, 0))
    return pl.pallas_call(
        _edge_body,
        grid=grid,
        in_specs=[
            ebs, ebs, ebs,
            pl.BlockSpec((HIDDEN, HEADS), lambda i: (0, 0)),
            pl.BlockSpec((HEADS, HIDDEN), lambda i: (0, 0)),
        ],
        out_specs=[ebs, ebs],
        out_shape=[jax.ShapeDtypeStruct((N_EDGES, HIDDEN), jnp.float32)] * 2,
    )(qd, ks, vs, S, SB)


def _scatter_body(w_hbm, dst_hbm, zw_hbm, pw_hbm,
                  dsti0, dsti1, wb0, wb1,
                  si0, si1, sw0, sw1, acc_w):
    cid = lax.axis_index("c")
    sid = lax.axis_index("s")
    wid = sid * NC + cid
    row0 = sid * ROWS_PER_TILE
    base0 = wid * EPW
    dsti = (dsti0, dsti1)
    wb = (wb0, wb1)
    si = (si0, si1)
    sw = (sw0, sw1)

    def start(i, b):
        pltpu.async_copy(dst_hbm.at[pl.ds(base0 + i * CH, CH)], dsti[b], si[b])
        pltpu.async_copy(w_hbm.at[pl.ds(base0 + i * CH, CH)], wb[b], sw[b])

    def wait(i, b):
        pltpu.make_async_copy(dst_hbm.at[pl.ds(base0 + i * CH, CH)],
                              dsti[b], si[b]).wait()
        pltpu.make_async_copy(w_hbm.at[pl.ds(base0 + i * CH, CH)],
                              wb[b], sw[b]).wait()

    def scatter(b):
        pltpu.sync_copy(wb[b], acc_w.at[dsti[b]], add=True)

    pltpu.sync_copy(zw_hbm, acc_w.at[pl.ds(row0, ROWS_PER_TILE)])
    plsc.subcore_barrier()

    start(0, 0)

    def body(g, carry):
        i0 = 2 * g
        wait(i0, 0)
        start(i0 + 1, 1)
        scatter(0)
        wait(i0 + 1, 1)

        @pl.when(i0 + 2 < NIT)
        def _():
            start(i0 + 2, 0)

        scatter(1)
        return carry

    lax.fori_loop(0, NIT // 2, body, 0)
    wait(NIT - 1, 0)
    scatter(0)

    plsc.subcore_barrier()
    pltpu.sync_copy(acc_w.at[pl.ds(row0, ROWS_PER_TILE)],
                    pw_hbm.at[cid, pl.ds(row0, ROWS_PER_TILE)])


@functools.lru_cache(maxsize=None)
def _build_scatter():
  return pl.kernel(
    _scatter_body,
    out_type=jax.ShapeDtypeStruct((NC, M_PAD, HIDDEN), jnp.float32),
    mesh=_sc_mesh(),
    scratch_types=[
        pltpu.VMEM((CH,), jnp.int32),
        pltpu.VMEM((CH,), jnp.int32),
        pltpu.VMEM((CH, HIDDEN), jnp.float32),
        pltpu.VMEM((CH, HIDDEN), jnp.float32),
        pltpu.SemaphoreType.DMA,
        pltpu.SemaphoreType.DMA,
        pltpu.SemaphoreType.DMA,
        pltpu.SemaphoreType.DMA,
        pltpu.VMEM_SHARED((M_PAD, HIDDEN), jnp.float32),
    ],
  )


def _out_body(h_ref, pw_ref, pe_ref, mod_ref,
              pjw_ref, pjb_ref, w1_ref, b1_ref, w2_ref, b2_ref, out_ref):
    num = pw_ref[0].astype(jnp.float32) + pw_ref[1].astype(jnp.float32)
    den = pe_ref[0].astype(jnp.float32) + pe_ref[1].astype(jnp.float32)
    msg = num / (den + 1e-9)
    attn = jnp.dot(msg, pjw_ref[...], preferred_element_type=jnp.float32) + pjb_ref[...]
    g_msa = mod_ref[0:1, 2 * HIDDEN:3 * HIDDEN]
    sh_mlp = mod_ref[0:1, 3 * HIDDEN:4 * HIDDEN]
    sc_mlp = mod_ref[0:1, 4 * HIDDEN:5 * HIDDEN]
    g_mlp = mod_ref[0:1, 5 * HIDDEN:6 * HIDDEN]
    h1 = h_ref[...] + g_msa * attn
    mu = jnp.mean(h1, axis=1, keepdims=True)
    var = jnp.mean((h1 - mu) * (h1 - mu), axis=1, keepdims=True)
    hm = (h1 - mu) * lax.rsqrt(var + 1e-6)
    hm = hm * (1.0 + sc_mlp) + sh_mlp
    z = jnp.dot(hm, w1_ref[...], preferred_element_type=jnp.float32) + b1_ref[...]
    t = 0.5 * z * (1.0 + lax.erf(z * np.float32(1.0 / np.sqrt(2.0))))
    mlp = jnp.dot(t, w2_ref[...], preferred_element_type=jnp.float32) + b2_ref[...]
    out_ref[...] = h1 + g_mlp * mlp


def _compute_out(h, pw, pe, mod_row, proj_w, proj_b, w1, b1, w2, b2):
    grid = (N_NODES // BM,)
    return pl.pallas_call(
        _out_body,
        grid=grid,
        in_specs=[
            pl.BlockSpec((BM, HIDDEN), lambda i: (i, 0)),
            pl.BlockSpec((NC, BM, HIDDEN), lambda i: (0, i, 0)),
            pl.BlockSpec((NC, BM, HIDDEN), lambda i: (0, i, 0)),
            pl.BlockSpec((1, 6 * HIDDEN), lambda i: (0, 0)),
            pl.BlockSpec((HIDDEN, HIDDEN), lambda i: (0, 0)),
            pl.BlockSpec((1, HIDDEN), lambda i: (0, 0)),
            pl.BlockSpec((HIDDEN, MLPD), lambda i: (0, 0)),
            pl.BlockSpec((1, MLPD), lambda i: (0, 0)),
            pl.BlockSpec((MLPD, HIDDEN), lambda i: (0, 0)),
            pl.BlockSpec((1, HIDDEN), lambda i: (0, 0)),
        ],
        out_specs=pl.BlockSpec((BM, HIDDEN), lambda i: (i, 0)),
        out_shape=jax.ShapeDtypeStruct((N_NODES, HIDDEN), jnp.float32),
    )(h, pw, pe, mod_row, proj_w, proj_b.reshape(1, HIDDEN),
      w1, b1.reshape(1, MLPD), w2, b2.reshape(1, HIDDEN))


def kernel(x, edge_index, forecast_delta, t_net_w, t_net_b, adaln_w, adaln_b,
           qkv_w, qkv_b, proj_w, proj_b, mlp_w1, mlp_b1, mlp_w2, mlp_b2):
    Bv, Nv, C = x.shape
    h = x.reshape(Bv * Nv, C)
    src = edge_index[0]
    dst = edge_index[1]

    lane = np.arange(HIDDEN) // DH
    S = jnp.asarray((lane[:, None] == np.arange(HEADS)[None, :]).astype(np.float32))
    SB = S.T
    zw = jnp.zeros((ROWS_PER_TILE, HIDDEN), jnp.float32)

    mod = _compute_mod(forecast_delta, t_net_w, t_net_b, adaln_w, adaln_b)

    for l in range(DEPTH):
        mod_row = mod[l:l + 1]
        q, k, v = _compute_qkv(h, mod_row, qkv_w[l], qkv_b[l])
        qd, ks, vs = _build_gather()(q, k, v, dst, src)
        w, eb = _compute_edge(qd, ks, vs, S, SB)
        pw = _build_scatter()(w, dst, zw)
        pe = _build_scatter()(eb, dst, zw)
        h = _compute_out(h, pw, pe, mod_row, proj_w[l], proj_b[l],
                         mlp_w1[l], mlp_b1[l], mlp_w2[l], mlp_b2[l])

    return h.reshape(Bv, Nv, C)


# speedup vs baseline: 45.3112x; 1.0068x over previous
import functools

import jax
import jax.numpy as jnp
import numpy as np
from jax import lax
from jax.experimental import pallas as pl
from jax.experimental.pallas import tpu as pltpu
from jax.experimental.pallas import tpu_sc as plsc

HIDDEN = 128
DEPTH = 4
HEADS = 8
DH = HIDDEN // HEADS
FREQ = 256
MLPD = 4 * HIDDEN

N_NODES = 10000
N_EDGES = 320000

BM = 2000
BE = 2000

NC = 2
NS = 16
NW = NC * NS
EPW = N_EDGES // NW
CH = 80
NIT = EPW // CH
M_PAD = 10240
ROWS_PER_TILE = M_PAD // NS


def _mod_body(fd_ref, tw_ref, tb_ref, aw_ref, ab_ref, out_ref):
    half = FREQ // 2
    t = fd_ref[0, 0]
    i = lax.broadcasted_iota(jnp.int32, (1, half), 1).astype(jnp.float32)
    freqs = jnp.exp(i * (-np.log(10000.0) / half))
    args = t * freqs
    emb = jnp.concatenate([jnp.cos(args), jnp.sin(args)], axis=1)
    temb = jnp.dot(emb, tw_ref[...], preferred_element_type=jnp.float32) + tb_ref[...]
    s = temb * jax.nn.sigmoid(temb)
    for l in range(DEPTH):
        row = jnp.dot(s, aw_ref[l], preferred_element_type=jnp.float32)
        out_ref[pl.ds(l, 1), :] = row + ab_ref[pl.ds(l, 1), :]


def _compute_mod(fd, t_net_w, t_net_b, adaln_w, adaln_b):
    return pl.pallas_call(
        _mod_body,
        out_shape=jax.ShapeDtypeStruct((DEPTH, 6 * HIDDEN), jnp.float32),
    )(fd.reshape(1, 1), t_net_w, t_net_b.reshape(1, HIDDEN), adaln_w, adaln_b)


def _qkv_body(h_ref, mod_ref, w_ref, b_ref, q_ref, kv_ref):
    h = h_ref[...]
    mu = jnp.mean(h, axis=1, keepdims=True)
    var = jnp.mean((h - mu) * (h - mu), axis=1, keepdims=True)
    hn = (h - mu) * lax.rsqrt(var + 1e-6)
    sh = mod_ref[0:1, 0:HIDDEN]
    sc = mod_ref[0:1, HIDDEN:2 * HIDDEN]
    hn = hn * (1.0 + sc) + sh
    qkv = jnp.dot(hn, w_ref[...], preferred_element_type=jnp.float32) + b_ref[...]
    q_ref[...] = qkv[:, 0:HIDDEN]
    kv_ref[...] = qkv[:, HIDDEN:3 * HIDDEN]


def _compute_qkv(h, mod_row, qkv_w, qkv_b):
    grid = (N_NODES // BM,)
    return pl.pallas_call(
        _qkv_body,
        grid=grid,
        in_specs=[
            pl.BlockSpec((BM, HIDDEN), lambda i: (i, 0)),
            pl.BlockSpec((1, 6 * HIDDEN), lambda i: (0, 0)),
            pl.BlockSpec((HIDDEN, 3 * HIDDEN), lambda i: (0, 0)),
            pl.BlockSpec((1, 3 * HIDDEN), lambda i: (0, 0)),
        ],
        out_specs=[pl.BlockSpec((BM, HIDDEN), lambda i: (i, 0)),
                   pl.BlockSpec((BM, 2 * HIDDEN), lambda i: (i, 0))],
        out_shape=[jax.ShapeDtypeStruct((N_NODES, HIDDEN), jnp.float32),
                   jax.ShapeDtypeStruct((N_NODES, 2 * HIDDEN), jnp.float32)],
    )(h, mod_row, qkv_w, qkv_b.reshape(1, 3 * HIDDEN))


@functools.lru_cache(maxsize=None)
def _sc_mesh():
    return plsc.VectorSubcoreMesh(
        core_axis_name="c", subcore_axis_name="s", num_cores=NC, num_subcores=NS)


def _gather_body(q_hbm, kv_hbm, dst_hbm, src_hbm,
                 qd_hbm, kvs_hbm,
                 di0, di1, sri0, sri1, qb0, qb1, kvb0, kvb1,
                 sdi0, sdi1, ssi0, ssi1,
                 sgq0, sgq1, sgk0, sgk1,
                 soq0, soq1, sok0, sok1):
    wid = lax.axis_index("s") * NC + lax.axis_index("c")
    base0 = wid * EPW
    di = (di0, di1)
    sri = (sri0, sri1)
    qb = (qb0, qb1)
    kvb = (kvb0, kvb1)
    sdi = (sdi0, sdi1)
    ssi = (ssi0, ssi1)
    sgq = (sgq0, sgq1)
    sgk = (sgk0, sgk1)
    soq = (soq0, soq1)
    sok = (sok0, sok1)

    def idx_start(i, b):
        pltpu.async_copy(dst_hbm.at[pl.ds(base0 + i * CH, CH)], di[b], sdi[b])
        pltpu.async_copy(src_hbm.at[pl.ds(base0 + i * CH, CH)], sri[b], ssi[b])

    def idx_wait(i, b):
        pltpu.make_async_copy(dst_hbm.at[pl.ds(base0 + i * CH, CH)],
                              di[b], sdi[b]).wait()
        pltpu.make_async_copy(src_hbm.at[pl.ds(base0 + i * CH, CH)],
                              sri[b], ssi[b]).wait()

    def g_start(b):
        pltpu.async_copy(q_hbm.at[di[b]], qb[b], sgq[b])
        pltpu.async_copy(kv_hbm.at[sri[b]], kvb[b], sgk[b])

    def g_wait(b):
        pltpu.make_async_copy(q_hbm.at[di[b]], qb[b], sgq[b]).wait()
        pltpu.make_async_copy(kv_hbm.at[sri[b]], kvb[b], sgk[b]).wait()

    def o_start(i, b):
        pltpu.async_copy(qb[b], qd_hbm.at[pl.ds(base0 + i * CH, CH)], soq[b])
        pltpu.async_copy(kvb[b], kvs_hbm.at[pl.ds(base0 + i * CH, CH)], sok[b])

    def o_wait(i, b):
        pltpu.make_async_copy(qb[b], qd_hbm.at[pl.ds(base0 + i * CH, CH)],
                              soq[b]).wait()
        pltpu.make_async_copy(kvb[b], kvs_hbm.at[pl.ds(base0 + i * CH, CH)],
                              sok[b]).wait()

    idx_start(0, 0)
    idx_start(1, 1)
    idx_wait(0, 0)
    g_start(0)

    def body(g, carry):
        i0 = 2 * g
        g_wait(0)
        o_start(i0, 0)
        idx_wait(i0 + 1, 1)

        @pl.when(i0 > 0)
        def _():
            o_wait(i0 - 1, 1)

        g_start(1)

        @pl.when(i0 + 2 < NIT)
        def _():
            idx_start(i0 + 2, 0)

        g_wait(1)

        @pl.when(i0 + 3 < NIT)
        def _():
            idx_start(i0 + 3, 1)

        @pl.when(i0 + 2 < NIT)
        def _():
            idx_wait(i0 + 2, 0)
            o_wait(i0, 0)
            g_start(0)

        o_start(i0 + 1, 1)
        return carry

    lax.fori_loop(0, NIT // 2, body, 0)
    g_wait(0)
    o_wait(NIT - 2, 1)
    o_start(NIT - 1, 0)
    o_wait(NIT - 1, 0)


@functools.lru_cache(maxsize=None)
def _build_gather():
  return pl.kernel(
    _gather_body,
    out_type=[jax.ShapeDtypeStruct((N_EDGES, HIDDEN), jnp.float32),
              jax.ShapeDtypeStruct((N_EDGES, 2 * HIDDEN), jnp.float32)],
    mesh=_sc_mesh(),
    scratch_types=[
        pltpu.VMEM((CH,), jnp.int32),
        pltpu.VMEM((CH,), jnp.int32),
        pltpu.VMEM((CH,), jnp.int32),
        pltpu.VMEM((CH,), jnp.int32),
        pltpu.VMEM((CH, HIDDEN), jnp.float32),
        pltpu.VMEM((CH, HIDDEN), jnp.float32),
        pltpu.VMEM((CH, 2 * HIDDEN), jnp.float32),
        pltpu.VMEM((CH, 2 * HIDDEN), jnp.float32),
    ] + [pltpu.SemaphoreType.DMA] * 12,
  )


def _edge_body(qd_ref, kvs_ref, S_ref, SB_ref, w_ref, e_ref):
    ks = kvs_ref[:, 0:HIDDEN]
    vs = kvs_ref[:, HIDDEN:2 * HIDDEN]
    p = qd_ref[...] * ks
    score = jnp.dot(p, S_ref[...], preferred_element_type=jnp.float32) * (1.0 / np.sqrt(DH))
    e = jnp.exp(score)
    eb = jnp.dot(e, SB_ref[...], preferred_element_type=jnp.float32)
    w_ref[...] = vs * eb
    e_ref[...] = eb


def _compute_edge(qd, kvs, S, SB):
    grid = (N_EDGES // BE,)
    ebs = pl.BlockSpec((BE, HIDDEN), lambda i: (i, 0))
    return pl.pallas_call(
        _edge_body,
        grid=grid,
        in_specs=[
            ebs,
            pl.BlockSpec((BE, 2 * HIDDEN), lambda i: (i, 0)),
            pl.BlockSpec((HIDDEN, HEADS), lambda i: (0, 0)),
            pl.BlockSpec((HEADS, HIDDEN), lambda i: (0, 0)),
        ],
        out_specs=[ebs, ebs],
        out_shape=[jax.ShapeDtypeStruct((N_EDGES, HIDDEN), jnp.float32)] * 2,
    )(qd, kvs, S, SB)


def _scatter_body(w_hbm, dst_hbm, zw_hbm, pw_hbm,
                  dsti0, dsti1, wb0, wb1,
                  si0, si1, sw0, sw1, acc_w):
    cid = lax.axis_index("c")
    sid = lax.axis_index("s")
    wid = sid * NC + cid
    row0 = sid * ROWS_PER_TILE
    base0 = wid * EPW
    dsti = (dsti0, dsti1)
    wb = (wb0, wb1)
    si = (si0, si1)
    sw = (sw0, sw1)

    def start(i, b):
        pltpu.async_copy(dst_hbm.at[pl.ds(base0 + i * CH, CH)], dsti[b], si[b])
        pltpu.async_copy(w_hbm.at[pl.ds(base0 + i * CH, CH)], wb[b], sw[b])

    def wait(i, b):
        pltpu.make_async_copy(dst_hbm.at[pl.ds(base0 + i * CH, CH)],
                              dsti[b], si[b]).wait()
        pltpu.make_async_copy(w_hbm.at[pl.ds(base0 + i * CH, CH)],
                              wb[b], sw[b]).wait()

    def scatter(b):
        pltpu.sync_copy(wb[b], acc_w.at[dsti[b]], add=True)

    pltpu.sync_copy(zw_hbm, acc_w.at[pl.ds(row0, ROWS_PER_TILE)])
    plsc.subcore_barrier()

    start(0, 0)

    def body(g, carry):
        i0 = 2 * g
        wait(i0, 0)
        start(i0 + 1, 1)
        scatter(0)
        wait(i0 + 1, 1)

        @pl.when(i0 + 2 < NIT)
        def _():
            start(i0 + 2, 0)

        scatter(1)
        return carry

    lax.fori_loop(0, NIT // 2, body, 0)
    wait(NIT - 1, 0)
    scatter(0)

    plsc.subcore_barrier()
    pltpu.sync_copy(acc_w.at[pl.ds(row0, ROWS_PER_TILE)],
                    pw_hbm.at[cid, pl.ds(row0, ROWS_PER_TILE)])


@functools.lru_cache(maxsize=None)
def _build_scatter():
  return pl.kernel(
    _scatter_body,
    out_type=jax.ShapeDtypeStruct((NC, M_PAD, HIDDEN), jnp.float32),
    mesh=_sc_mesh(),
    scratch_types=[
        pltpu.VMEM((CH,), jnp.int32),
        pltpu.VMEM((CH,), jnp.int32),
        pltpu.VMEM((CH, HIDDEN), jnp.float32),
        pltpu.VMEM((CH, HIDDEN), jnp.float32),
        pltpu.SemaphoreType.DMA,
        pltpu.SemaphoreType.DMA,
        pltpu.SemaphoreType.DMA,
        pltpu.SemaphoreType.DMA,
        pltpu.VMEM_SHARED((M_PAD, HIDDEN), jnp.float32),
    ],
  )


def _out_body(h_ref, pw_ref, pe_ref, mod_ref,
              pjw_ref, pjb_ref, w1_ref, b1_ref, w2_ref, b2_ref, out_ref):
    num = pw_ref[0].astype(jnp.float32) + pw_ref[1].astype(jnp.float32)
    den = pe_ref[0].astype(jnp.float32) + pe_ref[1].astype(jnp.float32)
    msg = num / (den + 1e-9)
    attn = jnp.dot(msg, pjw_ref[...], preferred_element_type=jnp.float32) + pjb_ref[...]
    g_msa = mod_ref[0:1, 2 * HIDDEN:3 * HIDDEN]
    sh_mlp = mod_ref[0:1, 3 * HIDDEN:4 * HIDDEN]
    sc_mlp = mod_ref[0:1, 4 * HIDDEN:5 * HIDDEN]
    g_mlp = mod_ref[0:1, 5 * HIDDEN:6 * HIDDEN]
    h1 = h_ref[...] + g_msa * attn
    mu = jnp.mean(h1, axis=1, keepdims=True)
    var = jnp.mean((h1 - mu) * (h1 - mu), axis=1, keepdims=True)
    hm = (h1 - mu) * lax.rsqrt(var + 1e-6)
    hm = hm * (1.0 + sc_mlp) + sh_mlp
    z = jnp.dot(hm, w1_ref[...], preferred_element_type=jnp.float32) + b1_ref[...]
    t = 0.5 * z * (1.0 + lax.erf(z * np.float32(1.0 / np.sqrt(2.0))))
    mlp = jnp.dot(t, w2_ref[...], preferred_element_type=jnp.float32) + b2_ref[...]
    out_ref[...] = h1 + g_mlp * mlp


def _compute_out(h, pw, pe, mod_row, proj_w, proj_b, w1, b1, w2, b2):
    grid = (N_NODES // BM,)
    return pl.pallas_call(
        _out_body,
        grid=grid,
        in_specs=[
            pl.BlockSpec((BM, HIDDEN), lambda i: (i, 0)),
            pl.BlockSpec((NC, BM, HIDDEN), lambda i: (0, i, 0)),
            pl.BlockSpec((NC, BM, HIDDEN), lambda i: (0, i, 0)),
            pl.BlockSpec((1, 6 * HIDDEN), lambda i: (0, 0)),
            pl.BlockSpec((HIDDEN, HIDDEN), lambda i: (0, 0)),
            pl.BlockSpec((1, HIDDEN), lambda i: (0, 0)),
            pl.BlockSpec((HIDDEN, MLPD), lambda i: (0, 0)),
            pl.BlockSpec((1, MLPD), lambda i: (0, 0)),
            pl.BlockSpec((MLPD, HIDDEN), lambda i: (0, 0)),
            pl.BlockSpec((1, HIDDEN), lambda i: (0, 0)),
        ],
        out_specs=pl.BlockSpec((BM, HIDDEN), lambda i: (i, 0)),
        out_shape=jax.ShapeDtypeStruct((N_NODES, HIDDEN), jnp.float32),
    )(h, pw, pe, mod_row, proj_w, proj_b.reshape(1, HIDDEN),
      w1, b1.reshape(1, MLPD), w2, b2.reshape(1, HIDDEN))


def kernel(x, edge_index, forecast_delta, t_net_w, t_net_b, adaln_w, adaln_b,
           qkv_w, qkv_b, proj_w, proj_b, mlp_w1, mlp_b1, mlp_w2, mlp_b2):
    Bv, Nv, C = x.shape
    h = x.reshape(Bv * Nv, C)
    src = edge_index[0]
    dst = edge_index[1]

    lane = np.arange(HIDDEN) // DH
    S = jnp.asarray((lane[:, None] == np.arange(HEADS)[None, :]).astype(np.float32))
    SB = S.T
    zw = jnp.zeros((ROWS_PER_TILE, HIDDEN), jnp.float32)

    mod = _compute_mod(forecast_delta, t_net_w, t_net_b, adaln_w, adaln_b)

    for l in range(DEPTH):
        mod_row = mod[l:l + 1]
        q, kv = _compute_qkv(h, mod_row, qkv_w[l], qkv_b[l])
        qd, kvs = _build_gather()(q, kv, dst, src)
        w, eb = _compute_edge(qd, kvs, S, SB)
        pw = _build_scatter()(w, dst, zw)
        pe = _build_scatter()(eb, dst, zw)
        h = _compute_out(h, pw, pe, mod_row, proj_w[l], proj_b[l],
                         mlp_w1[l], mlp_b1[l], mlp_w2[l], mlp_b2[l])

    return h.reshape(Bv, Nv, C)
